# baseline (speedup 1.0000x reference)
import functools

import jax
import jax.numpy as jnp
from jax import lax
from jax.experimental import pallas as pl
from jax.experimental.pallas import tpu as pltpu
from jax.experimental.pallas import tpu_sc as plsc


def _make_sc_gather(V, D, Btot, NC, NS):
    NW = NC * NS
    b_per_w = Btot // NW
    CHUNK = 128
    n_chunks = b_per_w // CHUNK
    mesh = plsc.VectorSubcoreMesh(core_axis_name="c", subcore_axis_name="s")

    @functools.partial(
        pl.kernel,
        mesh=mesh,
        compiler_params=pltpu.CompilerParams(use_tc_tiling_on_sc=False),
        out_type=[
            jax.ShapeDtypeStruct((Btot, D), jnp.float32),
            jax.ShapeDtypeStruct((Btot, D), jnp.float32),
        ],
        scratch_types=[
            pltpu.VMEM((b_per_w,), jnp.int32),
            pltpu.VMEM((b_per_w,), jnp.int32),
            pltpu.VMEM((b_per_w, D), jnp.float32),
            pltpu.VMEM((b_per_w, D), jnp.float32),
            pltpu.SemaphoreType.DMA,
        ],
    )
    def gather_k(emb_hbm, r_hbm, i_hbm, ro_hbm, io_hbm,
                 ridx, iidx, rrows, irows, sem):
        wid = lax.axis_index("s") * NC + lax.axis_index("c")
        base = wid * b_per_w
        pltpu.sync_copy(r_hbm.at[pl.ds(base, b_per_w)], ridx)
        pltpu.sync_copy(i_hbm.at[pl.ds(base, b_per_w)], iidx)
        copies = []
        for c in range(n_chunks):
            sl = pl.ds(c * CHUNK, CHUNK)
            copies.append(
                pltpu.async_copy(emb_hbm.at[ridx.at[sl]], rrows.at[sl], sem))
            copies.append(
                pltpu.async_copy(emb_hbm.at[iidx.at[sl]], irows.at[sl], sem))
        for cp in copies:
            cp.wait()
        pltpu.sync_copy(rrows, ro_hbm.at[pl.ds(base, b_per_w)])
        pltpu.sync_copy(irows, io_hbm.at[pl.ds(base, b_per_w)])

    return gather_k


def _linear_tanh(rrows, irows, w1t, w2t, b2d):
    B, D = rrows.shape
    blk = 2048

    def body(r_ref, i_ref, w1_ref, w2_ref, b_ref, o_ref):
        x = jnp.dot(r_ref[...], w1_ref[...],
                    preferred_element_type=jnp.float32)
        x = x + jnp.dot(i_ref[...], w2_ref[...],
                        preferred_element_type=jnp.float32)
        o_ref[...] = jnp.tanh(x + b_ref[...])

    return pl.pallas_call(
        body,
        grid=(B // blk,),
        in_specs=[
            pl.BlockSpec((blk, D), lambda i: (i, 0)),
            pl.BlockSpec((blk, D), lambda i: (i, 0)),
            pl.BlockSpec((D, D), lambda i: (0, 0)),
            pl.BlockSpec((D, D), lambda i: (0, 0)),
            pl.BlockSpec((1, D), lambda i: (0, 0)),
        ],
        out_specs=pl.BlockSpec((blk, D), lambda i: (i, 0)),
        out_shape=jax.ShapeDtypeStruct((B, D), jnp.float32),
    )(rrows, irows, w1t, w2t, b2d)


def kernel(ratings, items, emb, W, b):
    V, D = emb.shape
    (B,) = ratings.shape
    info = plsc.get_sparse_core_info()
    gather = _make_sc_gather(V, D, B, info.num_cores, info.num_subcores)
    rrows, irows = gather(emb, ratings.astype(jnp.int32),
                          items.astype(jnp.int32))
    wt = W.T
    out = _linear_tanh(rrows, irows, wt[:D], wt[D:], b[None, :])
    return out[None]
